# baseline (speedup 1.0000x reference)
import functools

import jax
import jax.numpy as jnp
from jax import lax
from jax.experimental import pallas as pl
from jax.experimental.pallas import tpu as pltpu
from jax.experimental.pallas import tpu_sc as plsc

N = 10000
D = 128
E = 320000
NW = 32
N_PAD = 10240
ROWS_W = N_PAD // NW
HALF = N_PAD // 2
SENT = HALF
N_RAW = HALF + 8
N_ACC = 10112
ASTRIPE = N_ACC // 16
JUNK = 10104
BLK = 32
NBLK = 648
E_PAD = 16 * NBLK * BLK


def _sc_mesh():
    return plsc.VectorSubcoreMesh(core_axis_name="c", subcore_axis_name="s")


def _emb_gather(vid_pad, emb_table):
    @functools.partial(
        pl.kernel,
        out_type=jax.ShapeDtypeStruct((N_PAD, D), jnp.float32),
        mesh=_sc_mesh(),
        scratch_types=[
            pltpu.VMEM((ROWS_W,), jnp.int32),
            pltpu.VMEM((ROWS_W, D), jnp.float32),
            pltpu.SemaphoreType.DMA,
        ],
    )
    def k(vid_hbm, emb_hbm, out_hbm, idx_v, rows_v, sem):
        wid = lax.axis_index("s") * 2 + lax.axis_index("c")
        base = wid * ROWS_W
        pltpu.sync_copy(vid_hbm.at[pl.ds(base, ROWS_W)], idx_v)
        pltpu.async_copy(emb_hbm.at[idx_v], rows_v, sem).wait()
        pltpu.sync_copy(rows_v, out_hbm.at[pl.ds(base, ROWS_W)])

    return k(vid_pad, emb_table)


def _edge_prop(raw_pad, eidx, zblk):
    @functools.partial(
        pl.kernel,
        out_type=jax.ShapeDtypeStruct((2, N_ACC, D), jnp.float32),
        mesh=_sc_mesh(),
        scratch_types=[
            [pltpu.VMEM((2, BLK), jnp.int32)] * 2,
            [pltpu.VMEM((BLK, D), jnp.float32)] * 2,
            pltpu.VMEM_SHARED((N_RAW, D), jnp.float32),
            pltpu.VMEM_SHARED((N_ACC, D), jnp.float32),
            [pltpu.SemaphoreType.DMA] * 2,
        ],
    )
    def k(raw_hbm, eidx_hbm, z_hbm, out_hbm, idxs, rows, raw_sh, acc_sh,
          sems):
        cid = lax.axis_index("c")
        sid = lax.axis_index("s")
        lo = cid * HALF

        pltpu.sync_copy(raw_hbm.at[pl.ds(lo + sid * ROWS_W, ROWS_W)],
                        raw_sh.at[pl.ds(sid * ROWS_W, ROWS_W)])

        @pl.when(sid == 15)
        def _():
            pltpu.sync_copy(z_hbm.at[pl.ds(0, 8)],
                            raw_sh.at[pl.ds(SENT, N_RAW - SENT)])

        pltpu.sync_copy(z_hbm, acc_sh.at[pl.ds(sid * ASTRIPE, ASTRIPE)])

        def fetch(j, b):
            pltpu.sync_copy(eidx_hbm.at[sid, j], idxs[b])
            for u in range(BLK // 16):
                s16 = idxs[b][0, pl.ds(u * 16, 16)]
                hit = (s16 >= lo) & (s16 < lo + HALF)
                idxs[b][0, pl.ds(u * 16, 16)] = jnp.where(hit, s16 - lo,
                                                          SENT)
            pltpu.async_copy(raw_sh.at[idxs[b].at[0]], rows[b], sems[b])

        plsc.subcore_barrier()
        for b in range(2):
            fetch(b, b)

        def body(t, carry):
            for b in range(2):
                j = 2 * t + b
                pltpu.make_async_copy(raw_sh.at[idxs[b].at[0]], rows[b],
                                      sems[b]).wait()
                pltpu.sync_copy(rows[b], acc_sh.at[idxs[b].at[1]], add=True)
                fetch(j + 2, b)
            return carry

        lax.fori_loop(0, NBLK // 2 - 1, body, 0)
        for b in range(2):
            pltpu.make_async_copy(raw_sh.at[idxs[b].at[0]], rows[b],
                                  sems[b]).wait()
            pltpu.sync_copy(rows[b], acc_sh.at[idxs[b].at[1]], add=True)

        plsc.subcore_barrier()
        pltpu.sync_copy(acc_sh.at[pl.ds(sid * ASTRIPE, ASTRIPE)],
                        out_hbm.at[cid, pl.ds(sid * ASTRIPE, ASTRIPE)])

    return k(raw_pad, eidx, zblk)


def _tc_head(raw_in, partials, labels2, W_self, W_nbr, b_gnn2, W_out, b_out2):
    def body(raw_ref, p_ref, lab_ref, ws_ref, wn_ref, bg_ref, wo_ref, bo_ref,
             logits_ref, loss_ref):
        raw = raw_ref[...]
        agg = p_ref[0] + p_ref[1]
        x = (jnp.dot(raw, ws_ref[...], preferred_element_type=jnp.float32)
             + jnp.dot(agg, wn_ref[...], preferred_element_type=jnp.float32)
             + bg_ref[...])
        x = jnp.maximum(x, 0.0)
        wo = wo_ref[...]
        logits = (jnp.dot(raw, wo[:D], preferred_element_type=jnp.float32)
                  + jnp.dot(x, wo[D:], preferred_element_type=jnp.float32)
                  + bo_ref[...])
        logits_ref[...] = logits
        m = jnp.max(logits, axis=-1, keepdims=True)
        lse = jnp.log(jnp.sum(jnp.exp(logits - m), axis=-1, keepdims=True)) + m
        cls = lax.broadcasted_iota(jnp.int32, logits.shape, 1)
        picked = jnp.sum(jnp.where(cls == lab_ref[...], logits, 0.0),
                         axis=-1, keepdims=True)
        loss_ref[...] = jnp.sum(lse - picked, axis=0, keepdims=True) / N

    return pl.pallas_call(
        body,
        out_shape=(
            jax.ShapeDtypeStruct((N, 10), jnp.float32),
            jax.ShapeDtypeStruct((1, 1), jnp.float32),
        ),
    )(raw_in, partials, labels2, W_self, W_nbr, b_gnn2, W_out, b_out2)


def kernel(vocab_ids, labels, edge_lists, emb_table, W_self, W_nbr, b_gnn,
           W_out, b_out):
    vid = vocab_ids.astype(jnp.int32)
    vid_pad = jnp.pad(vid, (0, N_PAD - N))
    raw_pad = _emb_gather(vid_pad, emb_table)

    src = edge_lists[0].astype(jnp.int32)
    dst = edge_lists[1].astype(jnp.int32)
    src_pad = jnp.pad(src, (0, E_PAD - E))
    dst_pad = jnp.pad(dst, (0, E_PAD - E), constant_values=JUNK)
    eidx = jnp.stack([src_pad.reshape(16, NBLK, BLK),
                      dst_pad.reshape(16, NBLK, BLK)], axis=2)
    zblk = jnp.zeros((ASTRIPE, D), jnp.float32)

    partials = _edge_prop(raw_pad, eidx, zblk)

    logits, loss2 = _tc_head(
        raw_pad[:N],
        partials[:, :N, :],
        labels.astype(jnp.int32).reshape(N, 1),
        W_self, W_nbr,
        b_gnn.reshape(1, D),
        W_out,
        b_out.reshape(1, 10),
    )
    return logits, loss2[0, 0]


# speedup vs baseline: 1.6480x; 1.6480x over previous
import functools

import jax
import jax.numpy as jnp
from jax import lax
from jax.experimental import pallas as pl
from jax.experimental.pallas import tpu as pltpu
from jax.experimental.pallas import tpu_sc as plsc

N = 10000
D = 128
E = 320000
NW = 32
N_PAD = 10240
ROWS_W = N_PAD // NW
HALF = N_PAD // 2
SENT = HALF
N_RAW = HALF + 8
N_ACC = 10112
ASTRIPE = N_ACC // 16
JUNK = 10104
BLK = 24
CPB = 8
NCHUNK = 108
NBLK = CPB * NCHUNK
E_PAD = 16 * NBLK * BLK


def _sc_mesh():
    return plsc.VectorSubcoreMesh(core_axis_name="c", subcore_axis_name="s")


def _emb_gather(vid_pad, emb_table):
    @functools.partial(
        pl.kernel,
        out_type=jax.ShapeDtypeStruct((N_PAD, D), jnp.float32),
        mesh=_sc_mesh(),
        scratch_types=[
            pltpu.VMEM((ROWS_W,), jnp.int32),
            pltpu.VMEM((ROWS_W, D), jnp.float32),
            pltpu.SemaphoreType.DMA,
        ],
    )
    def k(vid_hbm, emb_hbm, out_hbm, idx_v, rows_v, sem):
        wid = lax.axis_index("s") * 2 + lax.axis_index("c")
        base = wid * ROWS_W
        pltpu.sync_copy(vid_hbm.at[pl.ds(base, ROWS_W)], idx_v)
        pltpu.async_copy(emb_hbm.at[idx_v], rows_v, sem).wait()
        pltpu.sync_copy(rows_v, out_hbm.at[pl.ds(base, ROWS_W)])

    return k(vid_pad, emb_table)


def _edge_prop(raw_pad, src4, dst4, zblk):
    @functools.partial(
        pl.kernel,
        out_type=jax.ShapeDtypeStruct((2, N_ACC, D), jnp.float32),
        mesh=_sc_mesh(),
        scratch_types=[
            [pltpu.VMEM((CPB * BLK,), jnp.int32)] * 2,
            [pltpu.VMEM((CPB, BLK), jnp.int32)] * 2,
            [pltpu.VMEM((BLK, D), jnp.float32)] * 2,
            pltpu.VMEM_SHARED((N_RAW, D), jnp.float32),
            pltpu.VMEM_SHARED((N_ACC, D), jnp.float32),
            [pltpu.SemaphoreType.DMA] * 2,
            [pltpu.SemaphoreType.DMA] * 2,
            [pltpu.SemaphoreType.DMA] * 2,
        ],
    )
    def k(raw_hbm, src_hbm, dst_hbm, z_hbm, out_hbm, srcc, dstc, rows,
          raw_sh, acc_sh, gsems, ssems, dsems):
        cid = lax.axis_index("c")
        sid = lax.axis_index("s")
        lo = cid * HALF

        pltpu.sync_copy(raw_hbm.at[pl.ds(lo + sid * ROWS_W, ROWS_W)],
                        raw_sh.at[pl.ds(sid * ROWS_W, ROWS_W)])

        @pl.when(sid == 15)
        def _():
            pltpu.sync_copy(z_hbm.at[pl.ds(0, 8)],
                            raw_sh.at[pl.ds(SENT, N_RAW - SENT)])

        pltpu.sync_copy(z_hbm, acc_sh.at[pl.ds(sid * ASTRIPE, ASTRIPE)])

        def prefetch(c, cb):
            pltpu.async_copy(src_hbm.at[sid, c, 0], srcc[cb], ssems[cb])
            pltpu.async_copy(dst_hbm.at[sid, c], dstc[cb], dsems[cb])

        def mask_chunk(cb):
            for u in range(CPB * BLK // 16):
                s16 = srcc[cb][pl.ds(u * 16, 16)]
                hit = (s16 >= lo) & (s16 < lo + HALF)
                srcc[cb][pl.ds(u * 16, 16)] = jnp.where(hit, s16 - lo, SENT)

        def gather(cb, i, b):
            pltpu.async_copy(raw_sh.at[srcc[cb].at[pl.ds(i * BLK, BLK)]],
                             rows[b], gsems[b])

        def chunk(c, cb):
            pltpu.make_async_copy(src_hbm.at[sid, c, 0], srcc[cb],
                                  ssems[cb]).wait()
            pltpu.make_async_copy(dst_hbm.at[sid, c], dstc[cb],
                                  dsems[cb]).wait()

            @pl.when(c + 2 < NCHUNK)
            def _():
                prefetch(c + 2, 1 - cb)

            mask_chunk(cb)
            for i in range(2):
                gather(cb, i, i % 2)
            for i in range(CPB):
                b = i % 2
                pltpu.make_async_copy(
                    raw_sh.at[srcc[cb].at[pl.ds(i * BLK, BLK)]], rows[b],
                    gsems[b]).wait()
                pltpu.sync_copy(rows[b], acc_sh.at[dstc[cb].at[i]], add=True)
                if i + 2 < CPB:
                    gather(cb, i + 2, b)

        plsc.subcore_barrier()
        for cb in range(2):
            prefetch(cb, cb)

        def body(t, carry):
            for cb in range(2):
                chunk(2 * t + cb, cb)
            return carry

        lax.fori_loop(0, NCHUNK // 2, body, 0)

        plsc.subcore_barrier()
        pltpu.sync_copy(acc_sh.at[pl.ds(sid * ASTRIPE, ASTRIPE)],
                        out_hbm.at[cid, pl.ds(sid * ASTRIPE, ASTRIPE)])

    return k(raw_pad, src4, dst4, zblk)


def _tc_head(raw_in, partials, labels2, W_self, W_nbr, b_gnn2, W_out, b_out2):
    def body(raw_ref, p_ref, lab_ref, ws_ref, wn_ref, bg_ref, wo_ref, bo_ref,
             logits_ref, loss_ref):
        raw = raw_ref[...]
        agg = p_ref[0] + p_ref[1]
        x = (jnp.dot(raw, ws_ref[...], preferred_element_type=jnp.float32)
             + jnp.dot(agg, wn_ref[...], preferred_element_type=jnp.float32)
             + bg_ref[...])
        x = jnp.maximum(x, 0.0)
        wo = wo_ref[...]
        logits = (jnp.dot(raw, wo[:D], preferred_element_type=jnp.float32)
                  + jnp.dot(x, wo[D:], preferred_element_type=jnp.float32)
                  + bo_ref[...])
        logits_ref[...] = logits
        m = jnp.max(logits, axis=-1, keepdims=True)
        lse = jnp.log(jnp.sum(jnp.exp(logits - m), axis=-1, keepdims=True)) + m
        cls = lax.broadcasted_iota(jnp.int32, logits.shape, 1)
        picked = jnp.sum(jnp.where(cls == lab_ref[...], logits, 0.0),
                         axis=-1, keepdims=True)
        loss_ref[...] = jnp.sum(lse - picked, axis=0, keepdims=True) / N

    return pl.pallas_call(
        body,
        out_shape=(
            jax.ShapeDtypeStruct((N, 10), jnp.float32),
            jax.ShapeDtypeStruct((1, 1), jnp.float32),
        ),
    )(raw_in, partials, labels2, W_self, W_nbr, b_gnn2, W_out, b_out2)


def kernel(vocab_ids, labels, edge_lists, emb_table, W_self, W_nbr, b_gnn,
           W_out, b_out):
    vid = vocab_ids.astype(jnp.int32)
    vid_pad = jnp.pad(vid, (0, N_PAD - N))
    raw_pad = _emb_gather(vid_pad, emb_table)

    src = edge_lists[0].astype(jnp.int32)
    dst = edge_lists[1].astype(jnp.int32)
    src_pad = jnp.pad(src, (0, E_PAD - E))
    dst_pad = jnp.pad(dst, (0, E_PAD - E), constant_values=JUNK)
    src4 = src_pad.reshape(16, NCHUNK, 1, CPB * BLK)
    dst4 = dst_pad.reshape(16, NCHUNK, CPB, BLK)
    zblk = jnp.zeros((ASTRIPE, D), jnp.float32)

    partials = _edge_prop(raw_pad, src4, dst4, zblk)

    logits, loss2 = _tc_head(
        raw_pad[:N],
        partials[:, :N, :],
        labels.astype(jnp.int32).reshape(N, 1),
        W_self, W_nbr,
        b_gnn.reshape(1, D),
        W_out,
        b_out.reshape(1, 10),
    )
    return logits, loss2[0, 0]


# speedup vs baseline: 1.6489x; 1.0005x over previous
import functools

import jax
import jax.numpy as jnp
from jax import lax
from jax.experimental import pallas as pl
from jax.experimental.pallas import tpu as pltpu
from jax.experimental.pallas import tpu_sc as plsc

N = 10000
D = 128
E = 320000
NW = 32
N_PAD = 10240
ROWS_W = N_PAD // NW
HALF = N_PAD // 2
SENT = HALF
N_RAW = HALF + 8
N_ACC = 10112
ASTRIPE = N_ACC // 16
JUNK = 10104
BLK = 24
CPB = 8
NCHUNK = 108
NBLK = CPB * NCHUNK
E_PAD = 16 * NBLK * BLK


def _sc_mesh():
    return plsc.VectorSubcoreMesh(core_axis_name="c", subcore_axis_name="s")


def _emb_gather(vid_pad, emb_table):
    @functools.partial(
        pl.kernel,
        out_type=jax.ShapeDtypeStruct((N_PAD, D), jnp.float32),
        mesh=_sc_mesh(),
        scratch_types=[
            pltpu.VMEM((ROWS_W,), jnp.int32),
            pltpu.VMEM((ROWS_W, D), jnp.float32),
            pltpu.SemaphoreType.DMA,
        ],
    )
    def k(vid_hbm, emb_hbm, out_hbm, idx_v, rows_v, sem):
        wid = lax.axis_index("s") * 2 + lax.axis_index("c")
        base = wid * ROWS_W
        pltpu.sync_copy(vid_hbm.at[pl.ds(base, ROWS_W)], idx_v)
        pltpu.async_copy(emb_hbm.at[idx_v], rows_v, sem).wait()
        pltpu.sync_copy(rows_v, out_hbm.at[pl.ds(base, ROWS_W)])

    return k(vid_pad, emb_table)


def _edge_prop(raw_pad, src4, dst4, zblk):
    @functools.partial(
        pl.kernel,
        out_type=jax.ShapeDtypeStruct((2, N_ACC, D), jnp.float32),
        mesh=_sc_mesh(),
        scratch_types=[
            [pltpu.VMEM((CPB * BLK,), jnp.int32)] * 2,
            [pltpu.VMEM((CPB, BLK), jnp.int32)] * 2,
            [pltpu.VMEM((BLK, D), jnp.float32)] * 2,
            pltpu.VMEM_SHARED((N_RAW, D), jnp.float32),
            pltpu.VMEM_SHARED((N_ACC, D), jnp.float32),
            [pltpu.SemaphoreType.DMA] * 2,
            [pltpu.SemaphoreType.DMA] * 2,
            [pltpu.SemaphoreType.DMA] * 2,
        ],
    )
    def k(raw_hbm, src_hbm, dst_hbm, z_hbm, out_hbm, srcc, dstc, rows,
          raw_sh, acc_sh, gsems, ssems, dsems):
        cid = lax.axis_index("c")
        sid = lax.axis_index("s")
        lo = cid * HALF

        pltpu.sync_copy(raw_hbm.at[pl.ds(lo + sid * ROWS_W, ROWS_W)],
                        raw_sh.at[pl.ds(sid * ROWS_W, ROWS_W)])

        @pl.when(sid == 15)
        def _():
            pltpu.sync_copy(z_hbm.at[pl.ds(0, 8)],
                            raw_sh.at[pl.ds(SENT, N_RAW - SENT)])

        pltpu.sync_copy(z_hbm, acc_sh.at[pl.ds(sid * ASTRIPE, ASTRIPE)])

        def prefetch(c, cb):
            pltpu.async_copy(src_hbm.at[sid, c, 0], srcc[cb], ssems[cb])
            pltpu.async_copy(dst_hbm.at[sid, c], dstc[cb], dsems[cb])

        def mask_chunk(cb):
            for u in range(CPB * BLK // 16):
                s16 = srcc[cb][pl.ds(u * 16, 16)]
                hit = (s16 >= lo) & (s16 < lo + HALF)
                srcc[cb][pl.ds(u * 16, 16)] = jnp.where(hit, s16 - lo, SENT)

        def gather(cb, i, b):
            pltpu.async_copy(raw_sh.at[srcc[cb].at[pl.ds(i * BLK, BLK)]],
                             rows[b], gsems[b])

        def chunk(c, cb):
            pltpu.make_async_copy(src_hbm.at[sid, c, 0], srcc[cb],
                                  ssems[cb]).wait()
            pltpu.make_async_copy(dst_hbm.at[sid, c], dstc[cb],
                                  dsems[cb]).wait()
            mask_chunk(cb)
            for i in range(2):
                gather(cb, i, i % 2)
            for i in range(CPB):
                b = i % 2
                pltpu.make_async_copy(
                    raw_sh.at[srcc[cb].at[pl.ds(i * BLK, BLK)]], rows[b],
                    gsems[b]).wait()
                pltpu.sync_copy(rows[b], acc_sh.at[dstc[cb].at[i]], add=True)
                if i + 2 < CPB:
                    gather(cb, i + 2, b)

            @pl.when(c + 2 < NCHUNK)
            def _():
                prefetch(c + 2, cb)

        plsc.subcore_barrier()
        for cb in range(2):
            prefetch(cb, cb)

        def body(t, carry):
            for cb in range(2):
                chunk(2 * t + cb, cb)
            return carry

        lax.fori_loop(0, NCHUNK // 2, body, 0)

        plsc.subcore_barrier()
        pltpu.sync_copy(acc_sh.at[pl.ds(sid * ASTRIPE, ASTRIPE)],
                        out_hbm.at[cid, pl.ds(sid * ASTRIPE, ASTRIPE)])

    return k(raw_pad, src4, dst4, zblk)


def _tc_head(raw_in, partials, labels2, W_self, W_nbr, b_gnn2, W_out, b_out2):
    def body(raw_ref, p_ref, lab_ref, ws_ref, wn_ref, bg_ref, wo_ref, bo_ref,
             logits_ref, loss_ref):
        raw = raw_ref[...]
        agg = p_ref[0] + p_ref[1]
        x = (jnp.dot(raw, ws_ref[...], preferred_element_type=jnp.float32)
             + jnp.dot(agg, wn_ref[...], preferred_element_type=jnp.float32)
             + bg_ref[...])
        x = jnp.maximum(x, 0.0)
        wo = wo_ref[...]
        logits = (jnp.dot(raw, wo[:D], preferred_element_type=jnp.float32)
                  + jnp.dot(x, wo[D:], preferred_element_type=jnp.float32)
                  + bo_ref[...])
        logits_ref[...] = logits
        m = jnp.max(logits, axis=-1, keepdims=True)
        lse = jnp.log(jnp.sum(jnp.exp(logits - m), axis=-1, keepdims=True)) + m
        cls = lax.broadcasted_iota(jnp.int32, logits.shape, 1)
        picked = jnp.sum(jnp.where(cls == lab_ref[...], logits, 0.0),
                         axis=-1, keepdims=True)
        loss_ref[...] = jnp.sum(lse - picked, axis=0, keepdims=True) / N

    return pl.pallas_call(
        body,
        out_shape=(
            jax.ShapeDtypeStruct((N, 10), jnp.float32),
            jax.ShapeDtypeStruct((1, 1), jnp.float32),
        ),
    )(raw_in, partials, labels2, W_self, W_nbr, b_gnn2, W_out, b_out2)


def kernel(vocab_ids, labels, edge_lists, emb_table, W_self, W_nbr, b_gnn,
           W_out, b_out):
    vid = vocab_ids.astype(jnp.int32)
    vid_pad = jnp.pad(vid, (0, N_PAD - N))
    raw_pad = _emb_gather(vid_pad, emb_table)

    src = edge_lists[0].astype(jnp.int32)
    dst = edge_lists[1].astype(jnp.int32)
    src_pad = jnp.pad(src, (0, E_PAD - E))
    dst_pad = jnp.pad(dst, (0, E_PAD - E), constant_values=JUNK)
    src4 = src_pad.reshape(16, NCHUNK, 1, CPB * BLK)
    dst4 = dst_pad.reshape(16, NCHUNK, CPB, BLK)
    zblk = jnp.zeros((ASTRIPE, D), jnp.float32)

    partials = _edge_prop(raw_pad, src4, dst4, zblk)

    logits, loss2 = _tc_head(
        raw_pad[:N],
        partials[:, :N, :],
        labels.astype(jnp.int32).reshape(N, 1),
        W_self, W_nbr,
        b_gnn.reshape(1, D),
        W_out,
        b_out.reshape(1, 10),
    )
    return logits, loss2[0, 0]
